# baseline (speedup 1.0000x reference)
import functools

import jax
import jax.numpy as jnp
from jax import lax
from jax.experimental import pallas as pl
from jax.experimental.pallas import tpu as pltpu
from jax.experimental.pallas import tpu_sc as plsc

B, C, L = 4, 768, 1024
OUT = 159
OUTP = 256
EPS = 1e-5

NC, NS, LANES = 2, 16, 16
NW = NC * NS
EP = B * L // NW
FC = OUTP // LANES
LP = L + 8
TRASH = L


def _prep_body(x_ref, theta_ref, tbl_ref, gg1_ref, gg2_ref, ggs_ref,
               ll1_ref, ll2_ref, lls_ref):
    i = pl.program_id(0)
    xi = x_ref[0]

    sq = jnp.sum(xi * xi, axis=1, keepdims=True)
    g = lax.dot_general(xi, xi, (((1,), (1,)), ((), ())),
                        preferred_element_type=jnp.float32)
    d = sq - 2.0 * g + sq.T

    col = lax.broadcasted_iota(jnp.int32, (L, L), 1)
    m1 = jnp.min(d, axis=1, keepdims=True)
    a1 = jnp.min(jnp.where(d == m1, col, L), axis=1)
    d2 = jnp.where(col == a1[:, None], jnp.inf, d)
    m2 = jnp.min(d2, axis=1, keepdims=True)
    a2 = jnp.min(jnp.where(d2 == m2, col, L), axis=1)

    e_idx = lax.iota(jnp.int32, L)
    mself = jnp.logical_and(a1 != e_idx, a2 != e_idx)
    mi = mself.astype(jnp.int32)

    xt = jnp.dot(xi, theta_ref[...], preferred_element_type=jnp.float32)
    zpad = jnp.zeros((LP - L, OUTP), jnp.float32)
    tbl_ref[0, 0, :L] = xt * 0.5
    tbl_ref[0, 0, L:] = zpad
    tbl_ref[0, 1, :L] = xt * (1.0 / 3.0)
    tbl_ref[0, 1, L:] = zpad

    base = i * 2 * LP
    half = base + mi * LP
    gg1_ref[0, 0] = half + a1
    gg2_ref[0, 0] = half + a2
    ggs_ref[0, 0] = jnp.where(mself, base + LP + e_idx, base + L)

    ll1_ref[0, 0] = a1
    ll2_ref[0, 0] = a2
    lls_ref[0, 0] = jnp.where(mself, e_idx, TRASH)


def _agg_body(tbl_hbm, gg1_hbm, gg2_hbm, ggs_hbm, out_hbm,
              i1_v, i2_v, i3_v, r1_v, r2_v, r3_v, sem1, sem2, sem3):
    wid = lax.axis_index("s") * NC + lax.axis_index("c")
    ebase = wid * EP

    cps = [
        pltpu.async_copy(gg1_hbm.at[pl.ds(ebase, EP)], i1_v, sem1),
        pltpu.async_copy(gg2_hbm.at[pl.ds(ebase, EP)], i2_v, sem2),
        pltpu.async_copy(ggs_hbm.at[pl.ds(ebase, EP)], i3_v, sem3),
    ]
    for cp in cps:
        cp.wait()

    cp1 = pltpu.async_copy(tbl_hbm.at[i1_v], r1_v, sem1)
    cp2 = pltpu.async_copy(tbl_hbm.at[i2_v], r2_v, sem2)
    cp3 = pltpu.async_copy(tbl_hbm.at[i3_v], r3_v, sem3)
    cp1.wait()
    cp2.wait()
    cp3.wait()

    def _edge(j, _):
        for fc in range(FC):
            sl = pl.ds(fc * LANES, LANES)
            r1_v[j, sl] = r1_v[j, sl] + r2_v[j, sl] + r3_v[j, sl]
        return 0
    lax.fori_loop(0, EP, _edge, 0)

    pltpu.sync_copy(r1_v, out_hbm.at[pl.ds(ebase, EP)])


def _node_body(xe_ref, ll1_ref, ll2_ref, lls_ref, xn_ref):
    l1 = ll1_ref[0, 0]
    l2 = ll2_ref[0, 0]
    l3 = lls_ref[0, 0]
    col = lax.broadcasted_iota(jnp.int32, (L, L), 1)
    a = ((col == l1[:, None]) | (col == l2[:, None])
         | (col == l3[:, None])).astype(jnp.float32)
    deg = jnp.sum(a, axis=0)
    s = lax.dot_general(a, xe_ref[0], (((0,), (0,)), ((), ())),
                        preferred_element_type=jnp.float32)
    xn_ref[0] = s / deg[:, None]


def _bn_body(z_ref, gamma_ref, beta_ref, out_ref):
    z = z_ref[...]
    mean = jnp.mean(z, axis=(0, 2), keepdims=True)
    var = jnp.mean((z - mean) ** 2, axis=(0, 2), keepdims=True)
    y = (z - mean) * lax.rsqrt(var + EPS)
    y = y * gamma_ref[...][None, :, None] + beta_ref[...][None, :, None]
    out_ref[...] = jnp.maximum(y, 0.0)


@jax.jit
def kernel(x, theta, bn_gamma, bn_beta):
    xr = x.reshape(B, L, C)
    theta_pad = jnp.pad(theta, ((0, 0), (0, OUTP - OUT)))

    i32 = jnp.int32
    f32 = jnp.float32
    idx_spec = pl.BlockSpec((1, 1, L), lambda i: (i, 0, 0))
    idx_shape = jax.ShapeDtypeStruct((B, 1, L), i32)
    tbl, gg1, gg2, ggs, ll1, ll2, lls = pl.pallas_call(
        _prep_body,
        grid=(B,),
        in_specs=[
            pl.BlockSpec((1, L, C), lambda i: (i, 0, 0)),
            pl.BlockSpec((C, OUTP), lambda i: (0, 0)),
        ],
        out_specs=[
            pl.BlockSpec((1, 2, LP, OUTP), lambda i: (i, 0, 0, 0)),
            idx_spec, idx_spec, idx_spec, idx_spec, idx_spec, idx_spec,
        ],
        out_shape=[
            jax.ShapeDtypeStruct((B, 2, LP, OUTP), f32),
            idx_shape, idx_shape, idx_shape,
            idx_shape, idx_shape, idx_shape,
        ],
    )(xr, theta_pad)

    agg = pl.kernel(
        _agg_body,
        out_type=jax.ShapeDtypeStruct((B * L, OUTP), f32),
        mesh=plsc.VectorSubcoreMesh(core_axis_name="c", subcore_axis_name="s"),
        scratch_types=[
            pltpu.VMEM((EP,), i32),
            pltpu.VMEM((EP,), i32),
            pltpu.VMEM((EP,), i32),
            pltpu.VMEM((EP, OUTP), f32),
            pltpu.VMEM((EP, OUTP), f32),
            pltpu.VMEM((EP, OUTP), f32),
            pltpu.SemaphoreType.DMA,
            pltpu.SemaphoreType.DMA,
            pltpu.SemaphoreType.DMA,
        ],
    )
    xe = agg(tbl.reshape(B * 2 * LP, OUTP),
             gg1.reshape(B * L), gg2.reshape(B * L), ggs.reshape(B * L))

    xn = pl.pallas_call(
        _node_body,
        grid=(B,),
        in_specs=[
            pl.BlockSpec((1, L, OUTP), lambda i: (i, 0, 0)),
            idx_spec, idx_spec, idx_spec,
        ],
        out_specs=pl.BlockSpec((1, L, OUTP), lambda i: (i, 0, 0)),
        out_shape=jax.ShapeDtypeStruct((B, L, OUTP), f32),
    )(xe.reshape(B, L, OUTP), ll1, ll2, lls)

    z = xn.reshape(B * L, OUTP)[:, :OUT].reshape(B, OUT, L)
    y = pl.pallas_call(
        _bn_body,
        out_shape=jax.ShapeDtypeStruct((B, OUT, L), f32),
    )(z, bn_gamma, bn_beta)
    return y[..., None]


# speedup vs baseline: 1.4590x; 1.4590x over previous
import functools

import jax
import jax.numpy as jnp
from jax import lax
from jax.experimental import pallas as pl
from jax.experimental.pallas import tpu as pltpu
from jax.experimental.pallas import tpu_sc as plsc

B, C, L = 4, 768, 1024
OUT = 159
OUTP = 256
EPS = 1e-5

NC, NS, LANES = 2, 16, 16
NW = NC * NS
EP = B * L // NW
FC = OUT // LANES + 1
TRASH = L


def _prep_body(x_ref, theta_ref, tbl_ref, slf_ref, gg1_ref, gg2_ref,
               ll1_ref, ll2_ref, lls_ref):
    i = pl.program_id(0)
    xi = x_ref[0]

    sq = jnp.sum(xi * xi, axis=1, keepdims=True)
    g = lax.dot_general(xi, xi, (((1,), (1,)), ((), ())),
                        preferred_element_type=jnp.float32)
    d = sq - 2.0 * g + sq.T

    col = lax.broadcasted_iota(jnp.int32, (L, L), 1)
    m1 = jnp.min(d, axis=1, keepdims=True)
    a1 = jnp.min(jnp.where(d == m1, col, L), axis=1)
    d2 = jnp.where(col == a1[:, None], jnp.inf, d)
    m2 = jnp.min(d2, axis=1, keepdims=True)
    a2 = jnp.min(jnp.where(d2 == m2, col, L), axis=1)

    e_idx = lax.iota(jnp.int32, L)
    mself = jnp.logical_and(a1 != e_idx, a2 != e_idx)
    mi = mself.astype(jnp.int32)

    xt = jnp.dot(xi, theta_ref[...], preferred_element_type=jnp.float32)
    xt3 = xt * (1.0 / 3.0)
    tbl_ref[0, 0] = xt * 0.5
    tbl_ref[0, 1] = xt3
    slf_ref[0] = jnp.where(mself[:, None], xt3, 0.0)

    half = (i * 2 + mi) * L
    gg1_ref[0, 0] = half + a1
    gg2_ref[0, 0] = half + a2

    ll1_ref[0, 0] = a1
    ll2_ref[0, 0] = a2
    lls_ref[0, 0] = jnp.where(mself, e_idx, TRASH)


def _agg_body(tbl_hbm, slf_hbm, gg1_hbm, gg2_hbm, out_hbm,
              i1_v, i2_v, r1_v, r2_v, r3_v, sem1, sem2, sem3):
    wid = lax.axis_index("s") * NC + lax.axis_index("c")
    ebase = wid * EP

    cpa = pltpu.async_copy(gg1_hbm.at[pl.ds(ebase, EP)], i1_v, sem1)
    cpb = pltpu.async_copy(gg2_hbm.at[pl.ds(ebase, EP)], i2_v, sem2)
    cp3 = pltpu.async_copy(slf_hbm.at[pl.ds(ebase, EP)], r3_v, sem3)
    cpa.wait()
    cpb.wait()

    cp1 = pltpu.async_copy(tbl_hbm.at[i1_v], r1_v, sem1)
    cp2 = pltpu.async_copy(tbl_hbm.at[i2_v], r2_v, sem2)
    cp1.wait()
    cp2.wait()
    cp3.wait()

    def _edge(j, _):
        for fc in range(FC):
            sl = pl.ds(fc * LANES, LANES)
            r1_v[j, sl] = r1_v[j, sl] + r2_v[j, sl] + r3_v[j, sl]
        return 0
    lax.fori_loop(0, EP, _edge, 0)

    pltpu.sync_copy(r1_v, out_hbm.at[pl.ds(ebase, EP)])


def _node_body(xe_ref, ll1_ref, ll2_ref, lls_ref, xn_ref):
    l1 = ll1_ref[0, 0]
    l2 = ll2_ref[0, 0]
    l3 = lls_ref[0, 0]
    col = lax.broadcasted_iota(jnp.int32, (L, L), 1)
    a = ((col == l1[:, None]) | (col == l2[:, None])
         | (col == l3[:, None])).astype(jnp.float32)
    deg = jnp.sum(a, axis=0)
    s = lax.dot_general(a, xe_ref[0], (((0,), (0,)), ((), ())),
                        preferred_element_type=jnp.float32)
    xn_ref[0] = s / deg[:, None]


def _bn_body(z_ref, gamma_ref, beta_ref, out_ref):
    z = z_ref[...]
    mean = jnp.mean(z, axis=(0, 2), keepdims=True)
    var = jnp.mean((z - mean) ** 2, axis=(0, 2), keepdims=True)
    y = (z - mean) * lax.rsqrt(var + EPS)
    y = y * gamma_ref[...][None, :, None] + beta_ref[...][None, :, None]
    out_ref[...] = jnp.maximum(y, 0.0)


@jax.jit
def kernel(x, theta, bn_gamma, bn_beta):
    xr = x.reshape(B, L, C)
    theta_pad = jnp.pad(theta, ((0, 0), (0, OUTP - OUT)))

    i32 = jnp.int32
    f32 = jnp.float32
    idx_spec = pl.BlockSpec((1, 1, L), lambda i: (i, 0, 0))
    idx_shape = jax.ShapeDtypeStruct((B, 1, L), i32)
    tbl, slf, gg1, gg2, ll1, ll2, lls = pl.pallas_call(
        _prep_body,
        grid=(B,),
        in_specs=[
            pl.BlockSpec((1, L, C), lambda i: (i, 0, 0)),
            pl.BlockSpec((C, OUTP), lambda i: (0, 0)),
        ],
        out_specs=[
            pl.BlockSpec((1, 2, L, OUTP), lambda i: (i, 0, 0, 0)),
            pl.BlockSpec((1, L, OUTP), lambda i: (i, 0, 0)),
            idx_spec, idx_spec, idx_spec, idx_spec, idx_spec,
        ],
        out_shape=[
            jax.ShapeDtypeStruct((B, 2, L, OUTP), f32),
            jax.ShapeDtypeStruct((B, L, OUTP), f32),
            idx_shape, idx_shape, idx_shape, idx_shape, idx_shape,
        ],
    )(xr, theta_pad)

    agg = pl.kernel(
        _agg_body,
        out_type=jax.ShapeDtypeStruct((B * L, OUTP), f32),
        mesh=plsc.VectorSubcoreMesh(core_axis_name="c", subcore_axis_name="s"),
        scratch_types=[
            pltpu.VMEM((EP,), i32),
            pltpu.VMEM((EP,), i32),
            pltpu.VMEM((EP, OUTP), f32),
            pltpu.VMEM((EP, OUTP), f32),
            pltpu.VMEM((EP, OUTP), f32),
            pltpu.SemaphoreType.DMA,
            pltpu.SemaphoreType.DMA,
            pltpu.SemaphoreType.DMA,
        ],
    )
    xe = agg(tbl.reshape(B * 2 * L, OUTP), slf.reshape(B * L, OUTP),
             gg1.reshape(B * L), gg2.reshape(B * L))

    xn = pl.pallas_call(
        _node_body,
        grid=(B,),
        in_specs=[
            pl.BlockSpec((1, L, OUTP), lambda i: (i, 0, 0)),
            idx_spec, idx_spec, idx_spec,
        ],
        out_specs=pl.BlockSpec((1, L, OUTP), lambda i: (i, 0, 0)),
        out_shape=jax.ShapeDtypeStruct((B, L, OUTP), f32),
    )(xe.reshape(B, L, OUTP), ll1, ll2, lls)

    z = xn.reshape(B * L, OUTP)[:, :OUT].reshape(B, OUT, L)
    y = pl.pallas_call(
        _bn_body,
        out_shape=jax.ShapeDtypeStruct((B, OUT, L), f32),
    )(z, bn_gamma, bn_beta)
    return y[..., None]
